# baseline (speedup 1.0000x reference)
import jax
import jax.numpy as jnp
from jax import lax
from jax.experimental import pallas as pl
from jax.experimental.pallas import tpu as pltpu
from jax.experimental.pallas import tpu_sc as plsc

_B = 16384
_D = 32
_NE = 1000000
_NC = 2
_NS = 16
_DPC = _D // _NC
_IPT = _B // _NS
_NRV = 1000
_RV_LINES = _NRV * _D // 128
_NW = _NC * _NS
_BPW = _B // _NW
_R1 = 262144
_R2 = 524288
_R3 = 786432
_R4 = 999936
_LT = _NE - _R4


def _partial(Et_hbm, tailT_hbm, rv_hbm, u_hbm, r_hbm, v_hbm, part_hbm,
             row_a0, row_a1, row_b, row_c, row_d, row_t,
             uv_idx_v, uvA_v, uvB_v, uvC_v, uvD_v, uvT_v,
             r_idx_v, uv_val, acc, rv_l,
             sem_rv, sem_uv, sem_a0, sem_a1, sem_b, sem_c, sem_d, sem_t):
    cid = lax.axis_index("c")
    sid = lax.axis_index("s")
    base = sid * _IPT

    crv = pltpu.async_copy(rv_hbm, rv_l, sem_rv)
    pltpu.sync_copy(u_hbm.at[pl.ds(base, _IPT)],
                    uv_idx_v.at[pl.ds(0, _IPT)])
    pltpu.sync_copy(v_hbm.at[pl.ds(base, _IPT)],
                    uv_idx_v.at[pl.ds(_IPT, _IPT)])
    pltpu.sync_copy(r_hbm.at[pl.ds(base, _IPT)], r_idx_v)

    for s in range(2 * _IPT // 16):
        sl = pl.ds(s * 16, 16)
        neg1 = jnp.full((16,), -1, jnp.int32)
        r1 = jnp.full((16,), _R1, jnp.int32)
        r2 = jnp.full((16,), _R2, jnp.int32)
        r3 = jnp.full((16,), _R3, jnp.int32)
        r4 = jnp.full((16,), _R4, jnp.int32)
        i = uv_idx_v[sl]
        uvA_v[sl] = jnp.where(i < r1, i, neg1)
        uvB_v[sl] = jnp.where(jnp.logical_and(i >= r1, i < r2), i - r1, neg1)
        uvC_v[sl] = jnp.where(jnp.logical_and(i >= r2, i < r3), i - r2, neg1)
        uvD_v[sl] = jnp.where(jnp.logical_and(i >= r3, i < r4), i - r3, neg1)
        uvT_v[sl] = jnp.where(i >= r4, i - r4, neg1)

    for s in range(_IPT // 16):
        acc[pl.ds(s * 16, 16)] = jnp.zeros((16,), jnp.float32)
    crv.wait()

    three = jnp.full((16,), 3, jnp.int32)
    rows_a = (row_a0, row_a1)
    sems_a = (sem_a0, sem_a1)

    def fire_a(d, par):
        @pl.when(sid == 0)
        def _s():
            pltpu.async_copy(Et_hbm.at[d].at[pl.ds(0, _R1)],
                             rows_a[par], sems_a[par])

    def wait_a(d, par):
        @pl.when(sid == 0)
        def _w():
            pltpu.make_async_copy(Et_hbm.at[d].at[pl.ds(0, _R1)],
                                  rows_a[par], sems_a[par]).wait()

    def _single(buf, sem, lo, ln):
        def fire(d):
            @pl.when(sid == 0)
            def _s():
                pltpu.async_copy(Et_hbm.at[d].at[pl.ds(lo, ln)], buf, sem)

        def wait(d):
            @pl.when(sid == 0)
            def _w():
                pltpu.make_async_copy(
                    Et_hbm.at[d].at[pl.ds(lo, ln)], buf, sem).wait()

        return fire, wait

    fire_b, wait_b = _single(row_b, sem_b, _R1, _R2 - _R1)
    fire_c, wait_c = _single(row_c, sem_c, _R2, _R3 - _R2)
    fire_d, wait_d = _single(row_d, sem_d, _R3, _R4 - _R3)

    def fire_t(d):
        @pl.when(sid == 0)
        def _s():
            pltpu.async_copy(tailT_hbm.at[d], row_t, sem_t)

    def wait_t(d):
        @pl.when(sid == 0)
        def _w():
            pltpu.make_async_copy(tailT_hbm.at[d], row_t, sem_t).wait()

    d0 = cid * _DPC
    fire_a(d0, 0)
    fire_a(d0 + 1, 1)
    fire_b(d0)
    fire_c(d0)
    fire_d(d0)
    fire_t(d0)

    def pair_body(k2, carry):
        for par in range(2):
            k = k2 * 2 + par
            d = cid * _DPC + k
            wait_a(d, par)
            wait_b(d)
            wait_c(d)
            wait_d(d)
            wait_t(d)
            plsc.subcore_barrier()

            cps = [
                pltpu.async_copy(
                    rows_a[par].at[plsc.Indices(uvA_v, ignored_value=-1)],
                    uv_val, sem_uv),
                pltpu.async_copy(
                    row_b.at[plsc.Indices(uvB_v, ignored_value=-1)],
                    uv_val, sem_uv),
                pltpu.async_copy(
                    row_c.at[plsc.Indices(uvC_v, ignored_value=-1)],
                    uv_val, sem_uv),
                pltpu.async_copy(
                    row_d.at[plsc.Indices(uvD_v, ignored_value=-1)],
                    uv_val, sem_uv),
                pltpu.async_copy(
                    row_t.at[plsc.Indices(uvT_v, ignored_value=-1)],
                    uv_val, sem_uv),
            ]
            for c in cps:
                c.wait()
            plsc.subcore_barrier()

            @pl.when(k + 2 < _DPC)
            def _pf_a():
                fire_a(d + 2, par)

            @pl.when(k + 1 < _DPC)
            def _pf_rest():
                fire_b(d + 1)
                fire_c(d + 1)
                fire_d(d + 1)
                fire_t(d + 1)

            def slice_body(s, carry2, d=d):
                sl = pl.ds(s * 16, 16)
                ridx = r_idx_v[sl]
                rline = lax.shift_right_logical(ridx, 2)
                rcol = lax.shift_left(jnp.bitwise_and(ridx, three), 5) + d
                rd = plsc.load_gather(rv_l, [rline, rcol])
                t = uv_val[sl] - (uv_val[pl.ds(_IPT + s * 16, 16)] + rd)
                acc[sl] = acc[sl] + t * t
                return carry2

            lax.fori_loop(0, _IPT // 16, slice_body, 0)
        return carry

    lax.fori_loop(0, _DPC // 2, pair_body, 0)

    pltpu.sync_copy(acc, part_hbm.at[cid, pl.ds(base, _IPT)])


def _combine(part_hbm, out_hbm, p0_v, p1_v, out_v):
    wid = lax.axis_index("s") * _NC + lax.axis_index("c")
    base = wid * _BPW
    pltpu.sync_copy(part_hbm.at[0, pl.ds(base, _BPW)], p0_v)
    pltpu.sync_copy(part_hbm.at[1, pl.ds(base, _BPW)], p1_v)
    for s in range(_BPW // 16):
        sl = pl.ds(s * 16, 16)
        out_v[sl] = -(p0_v[sl] + p1_v[sl])
    pltpu.sync_copy(out_v, out_hbm.at[pl.ds(base, _BPW)])


@jax.jit
def kernel(E, rv, u_idx, r_idx, v_idx):
    Et = E.T
    tailT = lax.slice(Et, (0, _R4), (_D, _NE))
    rv_lines = rv.reshape(-1, 128)
    mesh = plsc.VectorSubcoreMesh(core_axis_name="c", subcore_axis_name="s")

    partial = pl.kernel(
        _partial,
        out_type=jax.ShapeDtypeStruct((_NC, _B), jnp.float32),
        mesh=mesh,
        compiler_params=pltpu.CompilerParams(needs_layout_passes=False),
        scratch_types=[
            pltpu.VMEM_SHARED((_R1,), jnp.float32),
            pltpu.VMEM_SHARED((_R1,), jnp.float32),
            pltpu.VMEM_SHARED((_R2 - _R1,), jnp.float32),
            pltpu.VMEM_SHARED((_R3 - _R2,), jnp.float32),
            pltpu.VMEM_SHARED((_R4 - _R3,), jnp.float32),
            pltpu.VMEM_SHARED((_LT,), jnp.float32),
            pltpu.VMEM((2 * _IPT,), jnp.int32),
            pltpu.VMEM((2 * _IPT,), jnp.int32),
            pltpu.VMEM((2 * _IPT,), jnp.int32),
            pltpu.VMEM((2 * _IPT,), jnp.int32),
            pltpu.VMEM((2 * _IPT,), jnp.int32),
            pltpu.VMEM((2 * _IPT,), jnp.int32),
            pltpu.VMEM((_IPT,), jnp.int32),
            pltpu.VMEM((2 * _IPT,), jnp.float32),
            pltpu.VMEM((_IPT,), jnp.float32),
            pltpu.VMEM((_RV_LINES, 128), jnp.float32),
            pltpu.SemaphoreType.DMA,
            pltpu.SemaphoreType.DMA,
            pltpu.SemaphoreType.DMA,
            pltpu.SemaphoreType.DMA,
            pltpu.SemaphoreType.DMA,
            pltpu.SemaphoreType.DMA,
            pltpu.SemaphoreType.DMA,
            pltpu.SemaphoreType.DMA,
        ],
    )
    part = partial(Et, tailT, rv_lines, u_idx, r_idx, v_idx)

    combine = pl.kernel(
        _combine,
        out_type=jax.ShapeDtypeStruct((_B,), jnp.float32),
        mesh=mesh,
        compiler_params=pltpu.CompilerParams(needs_layout_passes=False),
        scratch_types=[
            pltpu.VMEM((_BPW,), jnp.float32),
            pltpu.VMEM((_BPW,), jnp.float32),
            pltpu.VMEM((_BPW,), jnp.float32),
        ],
    )
    return combine(part)


# speedup vs baseline: 1.1862x; 1.1862x over previous
import jax
import jax.numpy as jnp
from jax import lax
from jax.experimental import pallas as pl
from jax.experimental.pallas import tpu as pltpu
from jax.experimental.pallas import tpu_sc as plsc

_B = 16384
_D = 32
_NE = 1000000
_NC = 2
_NS = 16
_DPC = _D // _NC
_IPT = _B // _NS
_NRV = 1000
_RV_LINES = _NRV * _D // 128
_NW = _NC * _NS
_BPW = _B // _NW


def _partial(Et_hbm, rv_hbm, u_hbm, r_hbm, v_hbm, part_hbm,
             row_sh, uv_idx_v, r_idx_v, uv_val, acc, rv_l,
             sem_rv, sem_uv, sem_uv2, sem_st):
    cid = lax.axis_index("c")
    sid = lax.axis_index("s")
    base = sid * _IPT

    crv = pltpu.async_copy(rv_hbm, rv_l, sem_rv)
    _H = _IPT // 2
    pltpu.sync_copy(u_hbm.at[pl.ds(base, _H)], uv_idx_v.at[pl.ds(0, _H)])
    pltpu.sync_copy(v_hbm.at[pl.ds(base, _H)], uv_idx_v.at[pl.ds(_H, _H)])
    pltpu.sync_copy(u_hbm.at[pl.ds(base + _H, _H)],
                    uv_idx_v.at[pl.ds(2 * _H, _H)])
    pltpu.sync_copy(v_hbm.at[pl.ds(base + _H, _H)],
                    uv_idx_v.at[pl.ds(3 * _H, _H)])
    pltpu.sync_copy(r_hbm.at[pl.ds(base, _IPT)], r_idx_v)

    for s in range(_IPT // 16):
        acc[pl.ds(s * 16, 16)] = jnp.zeros((16,), jnp.float32)
    crv.wait()

    three = jnp.full((16,), 3, jnp.int32)

    def fire_stage(d):
        @pl.when(sid == 0)
        def _stage():
            pltpu.async_copy(Et_hbm.at[d], row_sh, sem_st)

    def wait_stage(d):
        @pl.when(sid == 0)
        def _wait():
            pltpu.make_async_copy(Et_hbm.at[d], row_sh, sem_st).wait()

    fire_stage(cid * _DPC)

    def dim_body(k, carry):
        d = cid * _DPC + k
        wait_stage(d)
        plsc.subcore_barrier()

        h = _IPT // 2
        c0 = pltpu.async_copy(row_sh.at[uv_idx_v.at[pl.ds(0, 2 * h)]],
                              uv_val.at[pl.ds(0, 2 * h)], sem_uv)
        c1 = pltpu.async_copy(row_sh.at[uv_idx_v.at[pl.ds(2 * h, 2 * h)]],
                              uv_val.at[pl.ds(2 * h, 2 * h)], sem_uv2)

        def half_body(hb, u_off, b_off, d=d):
            def slice_body(s, carry2):
                sl = pl.ds(u_off + s * 16, 16)
                ridx = r_idx_v[pl.ds(b_off + s * 16, 16)]
                rline = lax.shift_right_logical(ridx, 2)
                rcol = lax.shift_left(jnp.bitwise_and(ridx, three), 5) + d
                rd = plsc.load_gather(rv_l, [rline, rcol])
                t = (uv_val[sl]
                     - (uv_val[pl.ds(u_off + h + s * 16, 16)] + rd))
                a_sl = pl.ds(b_off + s * 16, 16)
                acc[a_sl] = acc[a_sl] + t * t
                return carry2

            lax.fori_loop(0, h // 16, slice_body, 0)

        c0.wait()
        half_body(0, 0, 0)
        c1.wait()
        plsc.subcore_barrier()

        @pl.when(k + 1 < _DPC)
        def _prefetch():
            fire_stage(d + 1)

        half_body(1, 2 * h, h)
        return carry

    lax.fori_loop(0, _DPC, dim_body, 0)

    pltpu.sync_copy(acc, part_hbm.at[cid, pl.ds(base, _IPT)])


def _combine(part_hbm, out_hbm, p0_v, p1_v, out_v):
    wid = lax.axis_index("s") * _NC + lax.axis_index("c")
    base = wid * _BPW
    pltpu.sync_copy(part_hbm.at[0, pl.ds(base, _BPW)], p0_v)
    pltpu.sync_copy(part_hbm.at[1, pl.ds(base, _BPW)], p1_v)
    for s in range(_BPW // 16):
        sl = pl.ds(s * 16, 16)
        out_v[sl] = -(p0_v[sl] + p1_v[sl])
    pltpu.sync_copy(out_v, out_hbm.at[pl.ds(base, _BPW)])


@jax.jit
def kernel(E, rv, u_idx, r_idx, v_idx):
    Et = E.T
    rv_lines = rv.reshape(-1, 128)
    mesh = plsc.VectorSubcoreMesh(core_axis_name="c", subcore_axis_name="s")

    partial = pl.kernel(
        _partial,
        out_type=jax.ShapeDtypeStruct((_NC, _B), jnp.float32),
        mesh=mesh,
        compiler_params=pltpu.CompilerParams(needs_layout_passes=False),
        scratch_types=[
            pltpu.VMEM_SHARED((_NE,), jnp.float32),
            pltpu.VMEM((2 * _IPT,), jnp.int32),
            pltpu.VMEM((_IPT,), jnp.int32),
            pltpu.VMEM((2 * _IPT,), jnp.float32),
            pltpu.VMEM((_IPT,), jnp.float32),
            pltpu.VMEM((_RV_LINES, 128), jnp.float32),
            pltpu.SemaphoreType.DMA,
            pltpu.SemaphoreType.DMA,
            pltpu.SemaphoreType.DMA,
            pltpu.SemaphoreType.DMA,
        ],
    )
    part = partial(Et, rv_lines, u_idx, r_idx, v_idx)

    combine = pl.kernel(
        _combine,
        out_type=jax.ShapeDtypeStruct((_B,), jnp.float32),
        mesh=mesh,
        compiler_params=pltpu.CompilerParams(needs_layout_passes=False),
        scratch_types=[
            pltpu.VMEM((_BPW,), jnp.float32),
            pltpu.VMEM((_BPW,), jnp.float32),
            pltpu.VMEM((_BPW,), jnp.float32),
        ],
    )
    return combine(part)


# speedup vs baseline: 1.2929x; 1.0899x over previous
import jax
import jax.numpy as jnp
from jax import lax
from jax.experimental import pallas as pl
from jax.experimental.pallas import tpu as pltpu
from jax.experimental.pallas import tpu_sc as plsc

_B = 16384
_D = 32
_NE = 1000000
_NC = 2
_NS = 16
_DPC = _D // _NC
_IPT = _B // _NS
_NRV = 1000
_RV_LINES = _NRV * _D // 128
_NW = _NC * _NS
_BPW = _B // _NW


def _partial(Et_hbm, rv_hbm, u_hbm, r_hbm, v_hbm, part_hbm,
             row_sh, uv_idx_v, r_idx_v, uv_val, acc, rv_l,
             sem_rv, sem_uv, sem_st):
    cid = lax.axis_index("c")
    sid = lax.axis_index("s")
    base = sid * _IPT

    crv = pltpu.async_copy(rv_hbm, rv_l, sem_rv)
    pltpu.sync_copy(u_hbm.at[pl.ds(base, _IPT)],
                    uv_idx_v.at[pl.ds(0, _IPT)])
    pltpu.sync_copy(v_hbm.at[pl.ds(base, _IPT)],
                    uv_idx_v.at[pl.ds(_IPT, _IPT)])
    pltpu.sync_copy(r_hbm.at[pl.ds(base, _IPT)], r_idx_v)

    for s in range(_IPT // 16):
        acc[pl.ds(s * 16, 16)] = jnp.zeros((16,), jnp.float32)
    crv.wait()

    three = jnp.full((16,), 3, jnp.int32)

    def fire_stage(d):
        @pl.when(sid == 0)
        def _stage():
            pltpu.async_copy(Et_hbm.at[d], row_sh, sem_st)

    def wait_stage(d):
        @pl.when(sid == 0)
        def _wait():
            pltpu.make_async_copy(Et_hbm.at[d], row_sh, sem_st).wait()

    fire_stage(cid * _DPC)

    def dim_body(k, carry):
        d = cid * _DPC + k
        wait_stage(d)
        plsc.subcore_barrier()

        cuv = pltpu.async_copy(row_sh.at[uv_idx_v], uv_val, sem_uv)
        cuv.wait()
        plsc.subcore_barrier()

        @pl.when(k + 1 < _DPC)
        def _prefetch():
            fire_stage(d + 1)

        def slice_body(s, carry2, d=d):
            sl = pl.ds(s * 16, 16)
            ridx = r_idx_v[sl]
            rline = lax.shift_right_logical(ridx, 2)
            rcol = lax.shift_left(jnp.bitwise_and(ridx, three), 5) + d
            rd = plsc.load_gather(rv_l, [rline, rcol])
            t = uv_val[sl] - (uv_val[pl.ds(_IPT + s * 16, 16)] + rd)
            acc[sl] = acc[sl] + t * t
            return carry2

        lax.fori_loop(0, _IPT // 16, slice_body, 0)
        return carry

    lax.fori_loop(0, _DPC, dim_body, 0)

    pltpu.sync_copy(acc, part_hbm.at[cid, pl.ds(base, _IPT)])


def _combine(part_hbm, out_hbm, p0_v, p1_v, out_v):
    wid = lax.axis_index("s") * _NC + lax.axis_index("c")
    base = wid * _BPW
    pltpu.sync_copy(part_hbm.at[0, pl.ds(base, _BPW)], p0_v)
    pltpu.sync_copy(part_hbm.at[1, pl.ds(base, _BPW)], p1_v)
    for s in range(_BPW // 16):
        sl = pl.ds(s * 16, 16)
        out_v[sl] = -(p0_v[sl] + p1_v[sl])
    pltpu.sync_copy(out_v, out_hbm.at[pl.ds(base, _BPW)])


@jax.jit
def kernel(E, rv, u_idx, r_idx, v_idx):
    Et = E.T
    rv_lines = rv.reshape(-1, 128)
    mesh = plsc.VectorSubcoreMesh(core_axis_name="c", subcore_axis_name="s")

    partial = pl.kernel(
        _partial,
        out_type=jax.ShapeDtypeStruct((_NC, _B), jnp.float32),
        mesh=mesh,
        compiler_params=pltpu.CompilerParams(needs_layout_passes=False),
        scratch_types=[
            pltpu.VMEM_SHARED((_NE,), jnp.float32),
            pltpu.VMEM((2 * _IPT,), jnp.int32),
            pltpu.VMEM((_IPT,), jnp.int32),
            pltpu.VMEM((2 * _IPT,), jnp.float32),
            pltpu.VMEM((_IPT,), jnp.float32),
            pltpu.VMEM((_RV_LINES, 128), jnp.float32),
            pltpu.SemaphoreType.DMA,
            pltpu.SemaphoreType.DMA,
            pltpu.SemaphoreType.DMA,
        ],
    )
    part = partial(Et, rv_lines, u_idx, r_idx, v_idx)

    combine = pl.kernel(
        _combine,
        out_type=jax.ShapeDtypeStruct((_B,), jnp.float32),
        mesh=mesh,
        compiler_params=pltpu.CompilerParams(needs_layout_passes=False),
        scratch_types=[
            pltpu.VMEM((_BPW,), jnp.float32),
            pltpu.VMEM((_BPW,), jnp.float32),
            pltpu.VMEM((_BPW,), jnp.float32),
        ],
    )
    return combine(part)
